# baseline (speedup 1.0000x reference)
import functools

import jax
import jax.numpy as jnp
from jax import lax
from jax.experimental import pallas as pl
from jax.experimental.pallas import tpu as pltpu
from jax.experimental.pallas import tpu_sc as plsc

NC = 2
NS = 16
NW = NC * NS

K = 128
GPG = 8
ROWS_G = K * GPG


@functools.partial(jax.jit, static_argnames=("b", "d"))
def _gather_rows(x_flat, table, b, d):
    n_per_w = b // NW
    idx_rows_per_w = n_per_w // K
    n_groups = n_per_w // ROWS_G
    x2d = x_flat.reshape(b // K, K)

    mesh = plsc.VectorSubcoreMesh(core_axis_name="c", subcore_axis_name="s")

    @functools.partial(
        pl.kernel,
        out_type=jax.ShapeDtypeStruct((b, d), jnp.float32),
        mesh=mesh,
        scratch_types=[
            pltpu.VMEM((idx_rows_per_w, K), jnp.int32),
            pltpu.VMEM((ROWS_G, d), jnp.float32),
            pltpu.SemaphoreType.DMA,
        ],
        compiler_params=pltpu.CompilerParams(use_tc_tiling_on_sc=False),
    )
    def k(x_hbm, table_hbm, out_hbm, idx_v, rows_v, sem):
        wid = lax.axis_index("s") * NC + lax.axis_index("c")
        pltpu.sync_copy(
            x_hbm.at[pl.ds(wid * idx_rows_per_w, idx_rows_per_w)], idx_v
        )
        out_base = wid * n_per_w

        def group(g, carry):
            cps = []
            for j in range(GPG):
                cps.append(
                    pltpu.async_copy(
                        table_hbm.at[idx_v.at[g * GPG + j]],
                        rows_v.at[pl.ds(j * K, K)],
                        sem,
                    )
                )
            for cp in cps:
                cp.wait()
            pltpu.sync_copy(
                rows_v, out_hbm.at[pl.ds(out_base + g * ROWS_G, ROWS_G)]
            )
            return carry

        lax.fori_loop(0, n_groups, group, 0)

    return k(x2d, table)


def kernel(x, table):
    b, t = x.shape
    d = table.shape[1]
    x_flat = x.reshape(-1).astype(jnp.int32)
    out = _gather_rows(x_flat, table, b * t, d)
    return out.reshape(b, t, d)
